# baseline (speedup 1.0000x reference)
import jax
import jax.numpy as jnp
from jax import lax
from jax.experimental import pallas as pl
from jax.experimental.pallas import tpu as pltpu
from jax.experimental.pallas import tpu_sc as plsc

VOCAB = 100000
EMBED = 64
MAX_LEN = 200
BATCH = 4096

NUM_CORES = 2
NUM_SUBCORES = 16
NUM_WORKERS = NUM_CORES * NUM_SUBCORES
BBLK = BATCH // NUM_WORKERS
LANES = 16
RBLKS = BBLK // LANES


def _body(seq_hbm, table_hbm, pe_hbm, out_hbm, pe_v, idx_v,
          g0, g1, t0, t1, gs0, gs1, os0, os1):
    gbuf = (g0, g1)
    tbuf = (t0, t1)
    gsem = (gs0, gs1)
    osem = (os0, os1)
    wid = lax.axis_index("s") * NUM_CORES + lax.axis_index("c")
    b0 = wid * BBLK

    pltpu.sync_copy(pe_hbm, pe_v)
    pltpu.sync_copy(seq_hbm.at[:, pl.ds(b0, BBLK)], idx_v)

    lane_iota = lax.iota(jnp.int32, LANES)

    def gather_start(l, v):
        pltpu.make_async_copy(
            table_hbm.at[idx_v.at[l]], gbuf[v], gsem[v]).start()

    def gather_wait(l, v):
        pltpu.make_async_copy(
            table_hbm.at[idx_v.at[l]], gbuf[v], gsem[v]).wait()

    def out_start(l, v):
        dst = out_hbm.at[l, :, pl.ds(b0, BBLK)]
        pltpu.make_async_copy(tbuf[v], dst, osem[v]).start()

    def out_wait(l, v):
        dst = out_hbm.at[l, :, pl.ds(b0, BBLK)]
        pltpu.make_async_copy(tbuf[v], dst, osem[v]).wait()

    def transpose_add(l, v):
        def col(c, _):
            pe_c = plsc.load_gather(
                pe_v, [jnp.full((LANES,), c, jnp.int32),
                       jnp.full((LANES,), l, jnp.int32)])
            for rb in range(RBLKS):
                rows = lane_iota + (rb * LANES)
                vals = plsc.load_gather(
                    gbuf[v], [rows, jnp.full((LANES,), c, jnp.int32)])
                tbuf[v][c, pl.ds(rb * LANES, LANES)] = vals + pe_c
            return 0
        lax.fori_loop(0, EMBED, col, 0, unroll=2)

    gather_start(0, 0)

    def pair(p, _):
        for j in range(2):
            l = p * 2 + j
            v = j
            nv = 1 - j
            if j == 0:
                gather_start(l + 1, nv)
            else:
                @pl.when(p < MAX_LEN // 2 - 1)
                def _():
                    gather_start(l + 1, nv)
            gather_wait(l, v)
            @pl.when(p > 0)
            def _():
                out_wait(l - 2, v)
            transpose_add(l, v)
            out_start(l, v)
        return 0

    lax.fori_loop(0, MAX_LEN // 2, pair, 0)
    out_wait(MAX_LEN - 2, 0)
    out_wait(MAX_LEN - 1, 1)


@jax.jit
def _run(seq_t, token_table, pe_t):
    mesh = plsc.VectorSubcoreMesh(core_axis_name="c", subcore_axis_name="s")
    return pl.kernel(
        _body,
        out_type=jax.ShapeDtypeStruct((MAX_LEN, EMBED, BATCH), jnp.float32),
        mesh=mesh,
        compiler_params=pltpu.CompilerParams(
            use_tc_tiling_on_sc=False, needs_layout_passes=False),
        scratch_types=[
            pltpu.VMEM((EMBED, MAX_LEN), jnp.float32),
            pltpu.VMEM((MAX_LEN, BBLK), jnp.int32),
            pltpu.VMEM((BBLK, EMBED), jnp.float32),
            pltpu.VMEM((BBLK, EMBED), jnp.float32),
            pltpu.VMEM((EMBED, BBLK), jnp.float32),
            pltpu.VMEM((EMBED, BBLK), jnp.float32),
            pltpu.SemaphoreType.DMA,
            pltpu.SemaphoreType.DMA,
            pltpu.SemaphoreType.DMA,
            pltpu.SemaphoreType.DMA,
        ],
    )(seq_t, token_table, pe_t)


def kernel(sequence, token_table, pe_weight):
    seq_t = sequence.T.astype(jnp.int32)
    pe_t = pe_weight.T
    out_t = _run(seq_t, token_table, pe_t)
    return out_t.transpose(2, 0, 1)


# speedup vs baseline: 1.6737x; 1.6737x over previous
import jax
import jax.numpy as jnp
from jax import lax
from jax.experimental import pallas as pl
from jax.experimental.pallas import tpu as pltpu
from jax.experimental.pallas import tpu_sc as plsc

VOCAB = 100000
EMBED = 64
MAX_LEN = 200
BATCH = 4096

NUM_CORES = 2
NUM_SUBCORES = 16
NUM_WORKERS = NUM_CORES * NUM_SUBCORES
BBLK = BATCH // NUM_WORKERS
LANES = 16
RBLKS = BBLK // LANES


def _body(seq_hbm, table_hbm, pe_hbm, out_hbm, pe_v, idx_v,
          g0, g1, t0, t1, gs0, gs1, os0, os1):
    gbuf = (g0, g1)
    tbuf = (t0, t1)
    gsem = (gs0, gs1)
    osem = (os0, os1)
    wid = lax.axis_index("s") * NUM_CORES + lax.axis_index("c")
    b0 = wid * BBLK

    pltpu.sync_copy(pe_hbm, pe_v)
    pltpu.sync_copy(seq_hbm.at[:, pl.ds(b0, BBLK)], idx_v)

    lane_iota = lax.iota(jnp.int32, LANES)

    def gather_start(l, v):
        pltpu.make_async_copy(
            table_hbm.at[idx_v.at[l]], gbuf[v], gsem[v]).start()

    def gather_wait(l, v):
        pltpu.make_async_copy(
            table_hbm.at[idx_v.at[l]], gbuf[v], gsem[v]).wait()

    def out_start(l, v):
        dst = out_hbm.at[l, :, pl.ds(b0, BBLK)]
        pltpu.make_async_copy(tbuf[v], dst, osem[v]).start()

    def out_wait(l, v):
        dst = out_hbm.at[l, :, pl.ds(b0, BBLK)]
        pltpu.make_async_copy(tbuf[v], dst, osem[v]).wait()

    def transpose_add(l, v):
        @plsc.parallel_loop(0, EMBED, unroll=4)
        def col(c):
            pe_c = plsc.load_gather(
                pe_v, [jnp.full((LANES,), c, jnp.int32),
                       jnp.full((LANES,), l, jnp.int32)])
            for rb in range(RBLKS):
                rows = lane_iota + (rb * LANES)
                vals = plsc.load_gather(
                    gbuf[v], [rows, jnp.full((LANES,), c, jnp.int32)])
                tbuf[v][c, pl.ds(rb * LANES, LANES)] = vals + pe_c

    gather_start(0, 0)

    def pair(p, _):
        for j in range(2):
            l = p * 2 + j
            v = j
            nv = 1 - j
            if j == 0:
                gather_start(l + 1, nv)
            else:
                @pl.when(p < MAX_LEN // 2 - 1)
                def _():
                    gather_start(l + 1, nv)
            gather_wait(l, v)
            @pl.when(p > 0)
            def _():
                out_wait(l - 2, v)
            transpose_add(l, v)
            out_start(l, v)
        return 0

    lax.fori_loop(0, MAX_LEN // 2, pair, 0)
    out_wait(MAX_LEN - 2, 0)
    out_wait(MAX_LEN - 1, 1)


@jax.jit
def _run(seq_t, token_table, pe_t):
    mesh = plsc.VectorSubcoreMesh(core_axis_name="c", subcore_axis_name="s")
    return pl.kernel(
        _body,
        out_type=jax.ShapeDtypeStruct((MAX_LEN, EMBED, BATCH), jnp.float32),
        mesh=mesh,
        compiler_params=pltpu.CompilerParams(
            use_tc_tiling_on_sc=False, needs_layout_passes=False),
        scratch_types=[
            pltpu.VMEM((EMBED, MAX_LEN), jnp.float32),
            pltpu.VMEM((MAX_LEN, BBLK), jnp.int32),
            pltpu.VMEM((BBLK, EMBED), jnp.float32),
            pltpu.VMEM((BBLK, EMBED), jnp.float32),
            pltpu.VMEM((EMBED, BBLK), jnp.float32),
            pltpu.VMEM((EMBED, BBLK), jnp.float32),
            pltpu.SemaphoreType.DMA,
            pltpu.SemaphoreType.DMA,
            pltpu.SemaphoreType.DMA,
            pltpu.SemaphoreType.DMA,
        ],
    )(seq_t, token_table, pe_t)


def kernel(sequence, token_table, pe_weight):
    seq_t = sequence.T.astype(jnp.int32)
    pe_t = pe_weight.T
    out_t = _run(seq_t, token_table, pe_t)
    return out_t.transpose(2, 0, 1)


# speedup vs baseline: 1.7090x; 1.0211x over previous
import jax
import jax.numpy as jnp
from jax import lax
from jax.experimental import pallas as pl
from jax.experimental.pallas import tpu as pltpu
from jax.experimental.pallas import tpu_sc as plsc

VOCAB = 100000
EMBED = 64
MAX_LEN = 200
BATCH = 4096

NUM_CORES = 2
NUM_SUBCORES = 16
NUM_WORKERS = NUM_CORES * NUM_SUBCORES
N_BBLK = 16
BBLK = BATCH // N_BBLK
N_LHALF = NUM_WORKERS // N_BBLK
LHALF = MAX_LEN // N_LHALF
LANES = 16
RBLKS = BBLK // LANES


def _body(seq_hbm, table_hbm, pe_hbm, out_hbm, pe_v, idx_v,
          g0, g1, t0, t1, gs0, gs1, os0, os1):
    gbuf = (g0, g1)
    tbuf = (t0, t1)
    gsem = (gs0, gs1)
    osem = (os0, os1)
    wid = lax.axis_index("s") * NUM_CORES + lax.axis_index("c")
    b0 = (wid % N_BBLK) * BBLK
    l0 = (wid // N_BBLK) * LHALF

    pltpu.sync_copy(pe_hbm, pe_v)
    pltpu.sync_copy(seq_hbm.at[pl.ds(l0, LHALF), pl.ds(b0, BBLK)], idx_v)

    lane_iota = lax.iota(jnp.int32, LANES)

    def gather_start(i, v):
        pltpu.make_async_copy(
            table_hbm.at[idx_v.at[i]], gbuf[v], gsem[v]).start()

    def gather_wait(i, v):
        pltpu.make_async_copy(
            table_hbm.at[idx_v.at[i]], gbuf[v], gsem[v]).wait()

    def out_start(i, v):
        dst = out_hbm.at[l0 + i, :, pl.ds(b0, BBLK)]
        pltpu.make_async_copy(tbuf[v], dst, osem[v]).start()

    def out_wait(i, v):
        dst = out_hbm.at[l0 + i, :, pl.ds(b0, BBLK)]
        pltpu.make_async_copy(tbuf[v], dst, osem[v]).wait()

    def transpose_add(i, v):
        @plsc.parallel_loop(0, EMBED, unroll=2)
        def col(c):
            pe_c = plsc.load_gather(
                pe_v, [jnp.full((LANES,), c, jnp.int32),
                       jnp.full((LANES,), l0 + i, jnp.int32)])
            for rb in range(RBLKS):
                rows = lane_iota + (rb * LANES)
                vals = plsc.load_gather(
                    gbuf[v], [rows, jnp.full((LANES,), c, jnp.int32)])
                tbuf[v][c, pl.ds(rb * LANES, LANES)] = vals + pe_c

    gather_start(0, 0)

    def pair(p, _):
        for j in range(2):
            i = p * 2 + j
            v = j
            nv = 1 - j
            if j == 0:
                gather_start(i + 1, nv)
            else:
                @pl.when(p < LHALF // 2 - 1)
                def _():
                    gather_start(i + 1, nv)
            gather_wait(i, v)
            @pl.when(p > 0)
            def _():
                out_wait(i - 2, v)
            transpose_add(i, v)
            out_start(i, v)
        return 0

    lax.fori_loop(0, LHALF // 2, pair, 0)
    out_wait(LHALF - 2, 0)
    out_wait(LHALF - 1, 1)


@jax.jit
def _run(seq_t, token_table, pe_t):
    mesh = plsc.VectorSubcoreMesh(core_axis_name="c", subcore_axis_name="s")
    return pl.kernel(
        _body,
        out_type=jax.ShapeDtypeStruct((MAX_LEN, EMBED, BATCH), jnp.float32),
        mesh=mesh,
        compiler_params=pltpu.CompilerParams(
            use_tc_tiling_on_sc=False, needs_layout_passes=False),
        scratch_types=[
            pltpu.VMEM((EMBED, MAX_LEN), jnp.float32),
            pltpu.VMEM((LHALF, BBLK), jnp.int32),
            pltpu.VMEM((BBLK, EMBED), jnp.float32),
            pltpu.VMEM((BBLK, EMBED), jnp.float32),
            pltpu.VMEM((EMBED, BBLK), jnp.float32),
            pltpu.VMEM((EMBED, BBLK), jnp.float32),
            pltpu.SemaphoreType.DMA,
            pltpu.SemaphoreType.DMA,
            pltpu.SemaphoreType.DMA,
            pltpu.SemaphoreType.DMA,
        ],
    )(seq_t, token_table, pe_t)


def kernel(sequence, token_table, pe_weight):
    seq_t = sequence.T.astype(jnp.int32)
    pe_t = pe_weight.T
    out_t = _run(seq_t, token_table, pe_t)
    return out_t.transpose(2, 0, 1)


# speedup vs baseline: 3.5510x; 2.0778x over previous
import jax
import jax.numpy as jnp
from jax import lax
from jax.experimental import pallas as pl
from jax.experimental.pallas import tpu as pltpu
from jax.experimental.pallas import tpu_sc as plsc

VOCAB = 100000
EMBED = 64
MAX_LEN = 200
BATCH = 4096

NUM_CORES = 2
NUM_SUBCORES = 16
NUM_WORKERS = NUM_CORES * NUM_SUBCORES
N_BBLK = 16
BBLK = BATCH // N_BBLK
N_LHALF = NUM_WORKERS // N_BBLK
LHALF = MAX_LEN // N_LHALF
LANES = 16
RBLKS = BBLK // LANES
COLS = EMBED // LANES


def _body(seq_hbm, table_hbm, pe_hbm, out_hbm, pe_v, idx_v,
          g0, g1, g2, t0, t1, gs0, gs1, os0, os1):
    gbuf = (g0, g1)
    tbuf = (t0, t1)
    gsem = (gs0, gs1)
    osem = (os0, os1)
    wid = lax.axis_index("s") * NUM_CORES + lax.axis_index("c")
    b0 = (wid % N_BBLK) * BBLK
    l0 = (wid // N_BBLK) * LHALF

    pltpu.sync_copy(pe_hbm, pe_v)
    pltpu.sync_copy(seq_hbm.at[pl.ds(l0, LHALF), pl.ds(b0, BBLK)], idx_v)

    lane_iota = lax.iota(jnp.int32, LANES)

    def gather_start(i, v):
        pltpu.make_async_copy(
            table_hbm.at[idx_v.at[i]], gbuf[v], gsem[v]).start()

    def gather_wait(i, v):
        pltpu.make_async_copy(
            table_hbm.at[idx_v.at[i]], gbuf[v], gsem[v]).wait()

    def out_start(i, v):
        dst = out_hbm.at[l0 + i, :, pl.ds(b0, BBLK)]
        pltpu.make_async_copy(tbuf[v], dst, osem[v]).start()

    def out_wait(i, v):
        dst = out_hbm.at[l0 + i, :, pl.ds(b0, BBLK)]
        pltpu.make_async_copy(tbuf[v], dst, osem[v]).wait()

    def transpose_add(i, v):
        @plsc.parallel_loop(0, BBLK, unroll=4)
        def row(r):
            for cb in range(COLS):
                g2[r, pl.ds(cb * LANES, LANES)] = (
                    gbuf[v][r, pl.ds(cb * LANES, LANES)])

        @plsc.parallel_loop(0, EMBED, unroll=2)
        def col(c):
            pe_c = plsc.load_gather(
                pe_v, [jnp.full((LANES,), c, jnp.int32),
                       jnp.full((LANES,), l0 + i, jnp.int32)])
            for rb in range(RBLKS):
                rows = lane_iota + (rb * LANES)
                vals = plsc.load_gather(
                    g2, [rows, jnp.full((LANES,), c, jnp.int32)])
                tbuf[v][c, pl.ds(rb * LANES, LANES)] = vals + pe_c

    gather_start(0, 0)

    def pair(p, _):
        for j in range(2):
            i = p * 2 + j
            v = j
            nv = 1 - j
            if j == 0:
                gather_start(i + 1, nv)
            else:
                @pl.when(p < LHALF // 2 - 1)
                def _():
                    gather_start(i + 1, nv)
            gather_wait(i, v)
            @pl.when(p > 0)
            def _():
                out_wait(i - 2, v)
            transpose_add(i, v)
            out_start(i, v)
        return 0

    lax.fori_loop(0, LHALF // 2, pair, 0)
    out_wait(LHALF - 2, 0)
    out_wait(LHALF - 1, 1)


@jax.jit
def _run(seq_t, token_table, pe_t):
    mesh = plsc.VectorSubcoreMesh(core_axis_name="c", subcore_axis_name="s")
    return pl.kernel(
        _body,
        out_type=jax.ShapeDtypeStruct((MAX_LEN, EMBED, BATCH), jnp.float32),
        mesh=mesh,
        compiler_params=pltpu.CompilerParams(
            use_tc_tiling_on_sc=False, needs_layout_passes=False),
        scratch_types=[
            pltpu.VMEM((EMBED, MAX_LEN), jnp.float32),
            pltpu.VMEM((LHALF, BBLK), jnp.int32),
            pltpu.VMEM((BBLK, EMBED), jnp.float32),
            pltpu.VMEM((BBLK, EMBED), jnp.float32),
            pltpu.VMEM((BBLK, EMBED + 1), jnp.float32),
            pltpu.VMEM((EMBED, BBLK), jnp.float32),
            pltpu.VMEM((EMBED, BBLK), jnp.float32),
            pltpu.SemaphoreType.DMA,
            pltpu.SemaphoreType.DMA,
            pltpu.SemaphoreType.DMA,
            pltpu.SemaphoreType.DMA,
        ],
    )(seq_t, token_table, pe_t)


def kernel(sequence, token_table, pe_weight):
    seq_t = sequence.T.astype(jnp.int32)
    pe_t = pe_weight.T
    out_t = _run(seq_t, token_table, pe_t)
    return out_t.transpose(2, 0, 1)


# speedup vs baseline: 6.4781x; 1.8243x over previous
import jax
import jax.numpy as jnp
from jax import lax
from jax.experimental import pallas as pl
from jax.experimental.pallas import tpu as pltpu
from jax.experimental.pallas import tpu_sc as plsc

VOCAB = 100000
EMBED = 64
MAX_LEN = 200
BATCH = 4096

NUM_CORES = 2
NUM_SUBCORES = 16
NUM_WORKERS = NUM_CORES * NUM_SUBCORES
N_BBLK = 16
BBLK = BATCH // N_BBLK
N_LHALF = NUM_WORKERS // N_BBLK
LHALF = MAX_LEN // N_LHALF
LANES = 16
RBLKS = BBLK // LANES
COLS = EMBED // LANES


def _body(seq_hbm, table_hbm, pe_hbm, out_hbm, pe_v, idx_v,
          g0, g1, g2, t0, t1, gs0, gs1, os0, os1):
    gbuf = (g0, g1)
    tbuf = (t0, t1)
    gsem = (gs0, gs1)
    osem = (os0, os1)
    wid = lax.axis_index("s") * NUM_CORES + lax.axis_index("c")
    b0 = (wid % N_BBLK) * BBLK
    l0 = (wid // N_BBLK) * LHALF

    pltpu.sync_copy(pe_hbm, pe_v)
    pltpu.sync_copy(seq_hbm.at[pl.ds(l0, LHALF), pl.ds(b0, BBLK)], idx_v)

    lane_iota = lax.iota(jnp.int32, LANES)

    def gather_start(i, v):
        pltpu.make_async_copy(
            table_hbm.at[idx_v.at[i]], gbuf[v], gsem[v]).start()

    def gather_wait(i, v):
        pltpu.make_async_copy(
            table_hbm.at[idx_v.at[i]], gbuf[v], gsem[v]).wait()

    bh0 = (wid % N_BBLK) * (BBLK // 128)

    def out_start(i, v):
        dst = out_hbm.at[l0 + i, :, pl.ds(bh0, BBLK // 128)]
        pltpu.make_async_copy(tbuf[v], dst, osem[v]).start()

    def out_wait(i, v):
        dst = out_hbm.at[l0 + i, :, pl.ds(bh0, BBLK // 128)]
        pltpu.make_async_copy(tbuf[v], dst, osem[v]).wait()

    def transpose_add(i, v):
        @plsc.parallel_loop(0, BBLK, unroll=4)
        def row(r):
            for cb in range(COLS):
                g2[r, pl.ds(cb * LANES, LANES)] = (
                    gbuf[v][r, pl.ds(cb * LANES, LANES)])

        @plsc.parallel_loop(0, EMBED, unroll=2)
        def col(c):
            pe_c = plsc.load_gather(
                pe_v, [jnp.full((LANES,), c, jnp.int32),
                       jnp.full((LANES,), l0 + i, jnp.int32)])
            c_hi = c // 8
            c_lo = c % 8
            for rb in range(RBLKS):
                rows = lane_iota + (rb * LANES)
                vals = plsc.load_gather(
                    g2, [rows, jnp.full((LANES,), c, jnp.int32)])
                tbuf[v][c_hi, rb // 8, c_lo,
                        pl.ds((rb % 8) * LANES, LANES)] = vals + pe_c

    gather_start(0, 0)

    def pair(p, _):
        for j in range(2):
            i = p * 2 + j
            v = j
            nv = 1 - j
            if j == 0:
                gather_start(i + 1, nv)
            else:
                @pl.when(p < LHALF // 2 - 1)
                def _():
                    gather_start(i + 1, nv)
            gather_wait(i, v)
            @pl.when(p > 0)
            def _():
                out_wait(i - 2, v)
            transpose_add(i, v)
            out_start(i, v)
        return 0

    lax.fori_loop(0, LHALF // 2, pair, 0)
    out_wait(LHALF - 2, 0)
    out_wait(LHALF - 1, 1)


@jax.jit
def _run(seq_t, token_table, pe_t):
    mesh = plsc.VectorSubcoreMesh(core_axis_name="c", subcore_axis_name="s")
    return pl.kernel(
        _body,
        out_type=jax.ShapeDtypeStruct(
            (MAX_LEN, EMBED // 8, BATCH // 128, 8, 128), jnp.float32),
        mesh=mesh,
        compiler_params=pltpu.CompilerParams(
            use_tc_tiling_on_sc=False, needs_layout_passes=False),
        scratch_types=[
            pltpu.VMEM((EMBED, MAX_LEN), jnp.float32),
            pltpu.VMEM((LHALF, BBLK), jnp.int32),
            pltpu.VMEM((BBLK, EMBED), jnp.float32),
            pltpu.VMEM((BBLK, EMBED), jnp.float32),
            pltpu.VMEM((BBLK, EMBED + 1), jnp.float32),
            pltpu.VMEM((EMBED // 8, BBLK // 128, 8, 128),
                       jnp.float32),
            pltpu.VMEM((EMBED // 8, BBLK // 128, 8, 128),
                       jnp.float32),
            pltpu.SemaphoreType.DMA,
            pltpu.SemaphoreType.DMA,
            pltpu.SemaphoreType.DMA,
            pltpu.SemaphoreType.DMA,
        ],
    )(seq_t, token_table, pe_t)


def kernel(sequence, token_table, pe_weight):
    seq_t = sequence.T.astype(jnp.int32)
    pe_t = pe_weight.T
    out5 = _run(seq_t, token_table, pe_t)
    return out5.transpose(2, 4, 0, 1, 3).reshape(BATCH, MAX_LEN, EMBED)
